# baseline (speedup 1.0000x reference)
import jax
import jax.numpy as jnp
from jax.experimental import pallas as pl
from jax.experimental.pallas import tpu as pltpu

_NQ = 4
_K = 1024
_D = 256
_N = 16384
_BLK = 2048
_BETA = 0.25


def _vq_blk(x_ref, cb_ref, cbh_ref, cbm_ref, cbl_ref, xq_ref, idx_ref, lp_ref):
    x = x_ref[...]
    iota = jax.lax.broadcasted_iota(jnp.int32, (_BLK, _K), 1)
    r = x
    xq = jnp.zeros_like(x)
    lp = jnp.zeros((1, _D), jnp.float32)
    idx_cols = []
    for q in range(_NQ):
        cb = cb_ref[q]
        cb2 = jnp.sum(cb * cb, axis=1)
        r2 = jnp.sum(r * r, axis=1, keepdims=True)
        rb = r.astype(jnp.bfloat16)
        s = jax.lax.dot_general(rb, cbh_ref[q], (((1,), (1,)), ((), ())),
                                preferred_element_type=jnp.float32)
        d = (r2 + cb2[None, :]) - 2.0 * s
        m = jnp.min(d, axis=1, keepdims=True)
        idx = jnp.min(jnp.where(d == m, iota, _K), axis=1, keepdims=True)
        oh = (iota == idx).astype(jnp.bfloat16)
        z = ((jax.lax.dot_general(oh, cbh_ref[q], (((1,), (0,)), ((), ())),
                                  preferred_element_type=jnp.float32)
              + jax.lax.dot_general(oh, cbm_ref[q], (((1,), (0,)), ((), ())),
                                    preferred_element_type=jnp.float32))
             + jax.lax.dot_general(oh, cbl_ref[q], (((1,), (0,)), ((), ())),
                                   preferred_element_type=jnp.float32))
        diff = z - r
        lp = lp + jnp.sum(diff * diff, axis=0, keepdims=True)
        z_st = r + diff
        r = r - z_st
        xq = xq + z_st
        idx_cols.append(idx)
    xq_ref[...] = xq
    idx_ref[...] = jnp.concatenate(idx_cols, axis=1)
    lp_ref[...] = lp[None]


def _loss_fin(lp_ref, out_ref):
    s = jnp.sum(lp_ref[...], axis=(0, 1, 2), keepdims=True)
    out_ref[...] = s[0] * ((1.0 + _BETA) / (_NQ * _N * _D))


def _split_cb(cb_ref, hi_ref, mid_ref, lo_ref):
    cb = cb_ref[...]
    hi = cb.astype(jnp.bfloat16)
    d1 = cb - hi.astype(jnp.float32)
    mid = d1.astype(jnp.bfloat16)
    lo = (d1 - mid.astype(jnp.float32)).astype(jnp.bfloat16)
    hi_ref[...] = hi
    mid_ref[...] = mid
    lo_ref[...] = lo


def kernel(x, codebooks):
    cbh, cbm, cbl = pl.pallas_call(
        _split_cb,
        out_shape=[jax.ShapeDtypeStruct((_NQ, _K, _D), jnp.bfloat16)] * 3,
    )(codebooks)
    nb = _N // _BLK
    xq, idx, lp = pl.pallas_call(
        _vq_blk,
        grid=(nb,),
        in_specs=[
            pl.BlockSpec((_BLK, _D), lambda i: (i, 0)),
            pl.BlockSpec((_NQ, _K, _D), lambda i: (0, 0, 0)),
            pl.BlockSpec((_NQ, _K, _D), lambda i: (0, 0, 0)),
            pl.BlockSpec((_NQ, _K, _D), lambda i: (0, 0, 0)),
            pl.BlockSpec((_NQ, _K, _D), lambda i: (0, 0, 0)),
        ],
        out_specs=[
            pl.BlockSpec((_BLK, _D), lambda i: (i, 0)),
            pl.BlockSpec((_BLK, _NQ), lambda i: (i, 0)),
            pl.BlockSpec((1, 1, _D), lambda i: (i, 0, 0)),
        ],
        out_shape=[
            jax.ShapeDtypeStruct((_N, _D), jnp.float32),
            jax.ShapeDtypeStruct((_N, _NQ), jnp.int32),
            jax.ShapeDtypeStruct((nb, 1, _D), jnp.float32),
        ],
        compiler_params=pltpu.CompilerParams(
            dimension_semantics=("parallel",)),
    )(x, codebooks, cbh, cbm, cbl)
    loss = pl.pallas_call(
        _loss_fin,
        out_shape=jax.ShapeDtypeStruct((1, 1), jnp.float32),
    )(lp)
    return xq, loss.reshape(()), idx


# speedup vs baseline: 1.0036x; 1.0036x over previous
import jax
import jax.numpy as jnp
from jax.experimental import pallas as pl
from jax.experimental.pallas import tpu as pltpu

_NQ = 4
_K = 1024
_D = 256
_N = 16384
_BLK = 2048
_BETA = 0.25


def _vq_blk(x_ref, cbh_ref, w_ref, cb2_ref, xq_ref, idx_ref, lp_ref):
    x = x_ref[...]
    iota = jax.lax.broadcasted_iota(jnp.int32, (_BLK, _K), 1)
    r = x
    xq = jnp.zeros_like(x)
    lp = jnp.zeros((1, _D), jnp.float32)
    idx_cols = []
    for q in range(_NQ):
        cb2 = cb2_ref[q]
        r2 = jnp.sum(r * r, axis=1, keepdims=True)
        rb = r.astype(jnp.bfloat16)
        s = jax.lax.dot_general(rb, cbh_ref[q], (((1,), (1,)), ((), ())),
                                preferred_element_type=jnp.float32)
        d = (r2 + cb2) - 2.0 * s
        m = jnp.min(d, axis=1, keepdims=True)
        idx = jnp.min(jnp.where(d == m, iota, _K), axis=1, keepdims=True)
        oh = (iota == idx).astype(jnp.bfloat16)
        z3 = jax.lax.dot_general(oh, w_ref[q], (((1,), (0,)), ((), ())),
                                 preferred_element_type=jnp.float32)
        z = (z3[:, :_D] + z3[:, _D:2 * _D]) + z3[:, 2 * _D:]
        diff = z - r
        lp = lp + jnp.sum(diff * diff, axis=0, keepdims=True)
        z_st = r + diff
        r = r - z_st
        xq = xq + z_st
        idx_cols.append(idx)
    xq_ref[...] = xq
    idx_ref[...] = jnp.concatenate(idx_cols, axis=1)
    lp_ref[...] = lp[None]


def _loss_fin(lp_ref, out_ref):
    s = jnp.sum(lp_ref[...], axis=(0, 1, 2), keepdims=True)
    out_ref[...] = s[0] * ((1.0 + _BETA) / (_NQ * _N * _D))


def _split_cb(cb_ref, hi_ref, w_ref, cb2_ref):
    cb = cb_ref[...]
    hi = cb.astype(jnp.bfloat16)
    d1 = cb - hi.astype(jnp.float32)
    mid = d1.astype(jnp.bfloat16)
    lo = (d1 - mid.astype(jnp.float32)).astype(jnp.bfloat16)
    hi_ref[...] = hi
    w_ref[...] = jnp.concatenate([hi, mid, lo], axis=-1)
    cb2_ref[...] = jnp.sum(cb * cb, axis=-1)[:, None, :]


def kernel(x, codebooks):
    cbh, w, cb2 = pl.pallas_call(
        _split_cb,
        out_shape=[jax.ShapeDtypeStruct((_NQ, _K, _D), jnp.bfloat16),
                   jax.ShapeDtypeStruct((_NQ, _K, 3 * _D), jnp.bfloat16),
                   jax.ShapeDtypeStruct((_NQ, 1, _K), jnp.float32)],
    )(codebooks)
    nb = _N // _BLK
    xq, idx, lp = pl.pallas_call(
        _vq_blk,
        grid=(nb,),
        in_specs=[
            pl.BlockSpec((_BLK, _D), lambda i: (i, 0)),
            pl.BlockSpec((_NQ, _K, _D), lambda i: (0, 0, 0)),
            pl.BlockSpec((_NQ, _K, 3 * _D), lambda i: (0, 0, 0)),
            pl.BlockSpec((_NQ, 1, _K), lambda i: (0, 0, 0)),
        ],
        out_specs=[
            pl.BlockSpec((_BLK, _D), lambda i: (i, 0)),
            pl.BlockSpec((_BLK, _NQ), lambda i: (i, 0)),
            pl.BlockSpec((1, 1, _D), lambda i: (i, 0, 0)),
        ],
        out_shape=[
            jax.ShapeDtypeStruct((_N, _D), jnp.float32),
            jax.ShapeDtypeStruct((_N, _NQ), jnp.int32),
            jax.ShapeDtypeStruct((nb, 1, _D), jnp.float32),
        ],
        compiler_params=pltpu.CompilerParams(
            dimension_semantics=("parallel",)),
    )(x, cbh, w, cb2)
    loss = pl.pallas_call(
        _loss_fin,
        out_shape=jax.ShapeDtypeStruct((1, 1), jnp.float32),
    )(lp)
    return xq, loss.reshape(()), idx


# speedup vs baseline: 1.9416x; 1.9346x over previous
import jax
import jax.numpy as jnp
from jax.experimental import pallas as pl
from jax.experimental.pallas import tpu as pltpu

_NQ = 4
_K = 1024
_D = 256
_N = 16384
_BLK = 2048
_BETA = 0.25


_H = _BLK // 2


def _vq_blk(x_ref, cbh_ref, w_ref, cb2_ref, xq_ref, idx_ref, lp_ref):
    iota = jax.lax.broadcasted_iota(jnp.int32, (_H, _K), 1)
    r = [x_ref[:_H, :], x_ref[_H:, :]]
    xq = [jnp.zeros((_H, _D), jnp.float32) for _ in range(2)]
    lp = [jnp.zeros((1, _D), jnp.float32) for _ in range(2)]
    idx_cols = [[], []]
    for q in range(_NQ):
        cb2 = cb2_ref[q]
        s = [None, None]
        for h in range(2):
            rb = r[h].astype(jnp.bfloat16)
            s[h] = jax.lax.dot_general(rb, cbh_ref[q], (((1,), (1,)), ((), ())),
                                       preferred_element_type=jnp.float32)
        oh = [None, None]
        for h in range(2):
            r2 = jnp.sum(r[h] * r[h], axis=1, keepdims=True)
            d = (r2 + cb2) - 2.0 * s[h]
            m = jnp.min(d, axis=1, keepdims=True)
            idx = jnp.min(jnp.where(d == m, iota, _K), axis=1, keepdims=True)
            oh[h] = (iota == idx).astype(jnp.bfloat16)
            idx_cols[h].append(idx)
        for h in range(2):
            z3 = jax.lax.dot_general(oh[h], w_ref[q], (((1,), (0,)), ((), ())),
                                     preferred_element_type=jnp.float32)
            z = (z3[:, :_D] + z3[:, _D:2 * _D]) + z3[:, 2 * _D:]
            diff = z - r[h]
            lp[h] = lp[h] + jnp.sum(diff * diff, axis=0, keepdims=True)
            z_st = r[h] + diff
            r[h] = r[h] - z_st
            xq[h] = xq[h] + z_st
    xq_ref[:_H, :] = xq[0]
    xq_ref[_H:, :] = xq[1]
    idx_ref[:_H, :] = jnp.concatenate(idx_cols[0], axis=1)
    idx_ref[_H:, :] = jnp.concatenate(idx_cols[1], axis=1)
    lp_ref[...] = (lp[0] + lp[1])[None]


def _loss_fin(lp_ref, out_ref):
    s = jnp.sum(lp_ref[...], axis=(0, 1, 2), keepdims=True)
    out_ref[...] = s[0] * ((1.0 + _BETA) / (_NQ * _N * _D))


def _split_cb(cb_ref, hi_ref, w_ref, cb2_ref):
    cb = cb_ref[...]
    hi = cb.astype(jnp.bfloat16)
    d1 = cb - hi.astype(jnp.float32)
    mid = d1.astype(jnp.bfloat16)
    lo = (d1 - mid.astype(jnp.float32)).astype(jnp.bfloat16)
    hi_ref[...] = hi
    w_ref[...] = jnp.concatenate([hi, mid, lo], axis=-1)
    cb2_ref[...] = jnp.sum(cb * cb, axis=-1)[:, None, :]


def kernel(x, codebooks):
    cbh, w, cb2 = pl.pallas_call(
        _split_cb,
        out_shape=[jax.ShapeDtypeStruct((_NQ, _K, _D), jnp.bfloat16),
                   jax.ShapeDtypeStruct((_NQ, _K, 3 * _D), jnp.bfloat16),
                   jax.ShapeDtypeStruct((_NQ, 1, _K), jnp.float32)],
    )(codebooks)
    nb = _N // _BLK
    xq, idx, lp = pl.pallas_call(
        _vq_blk,
        grid=(nb,),
        in_specs=[
            pl.BlockSpec((_BLK, _D), lambda i: (i, 0)),
            pl.BlockSpec((_NQ, _K, _D), lambda i: (0, 0, 0)),
            pl.BlockSpec((_NQ, _K, 3 * _D), lambda i: (0, 0, 0)),
            pl.BlockSpec((_NQ, 1, _K), lambda i: (0, 0, 0)),
        ],
        out_specs=[
            pl.BlockSpec((_BLK, _D), lambda i: (i, 0)),
            pl.BlockSpec((_BLK, _NQ), lambda i: (i, 0)),
            pl.BlockSpec((1, 1, _D), lambda i: (i, 0, 0)),
        ],
        out_shape=[
            jax.ShapeDtypeStruct((_N, _D), jnp.float32),
            jax.ShapeDtypeStruct((_N, _NQ), jnp.int32),
            jax.ShapeDtypeStruct((nb, 1, _D), jnp.float32),
        ],
        compiler_params=pltpu.CompilerParams(
            dimension_semantics=("parallel",)),
    )(x, cbh, w, cb2)
    loss = pl.pallas_call(
        _loss_fin,
        out_shape=jax.ShapeDtypeStruct((1, 1), jnp.float32),
    )(lp)
    return xq, loss.reshape(()), idx
